# baseline (speedup 1.0000x reference)
import functools

import jax
import jax.numpy as jnp
from jax import lax
from jax.experimental import pallas as pl
from jax.experimental.pallas import tpu as pltpu
from jax.experimental.pallas import tpu_sc as plsc

N_FIELDS = 26
VOCAB = 100000
EMBED_DIM = 16
BATCH = 16384
D_CON = 13
D_OUT = D_CON + N_FIELDS * EMBED_DIM

NC, NS, L = 2, 16, 16
F_PER_SC = N_FIELDS // NC
BW = BATCH // NS
STRIPE = 6256
V0_CAP = VOCAB - STRIPE
GSUB = 128
EQ = 4
N_Q = EMBED_DIM // EQ


def _body(x_con_hbm, x_cat_hbm, tables_hbm, out_hbm,
          idx_v, fidx_v, outb_v, slab_a, slab_b,
          sem_a, sem_b, idx_sem, g_sem, wr_sem):
    sc = lax.axis_index("c")
    t = lax.axis_index("s")
    col0 = t * BW
    stripe_v0 = pl.multiple_of(jnp.minimum(t * STRIPE, V0_CAP), 8)
    f0 = sc * F_PER_SC

    pltpu.sync_copy(x_con_hbm.at[:, pl.ds(col0, BW)],
                    outb_v.at[pl.ds(0, D_CON)])
    pltpu.sync_copy(outb_v.at[pl.ds(0, D_CON)],
                    out_hbm.at[pl.ds(0, D_CON), pl.ds(col0, BW)])

    def stage_quarter(i, q, slot, sem):
        for e in range(EQ):
            pltpu.async_copy(
                tables_hbm.at[i, q * EQ + e, pl.ds(stripe_v0, STRIPE)],
                slot.at[pl.ds(e * VOCAB + stripe_v0, STRIPE)],
                sem)

    def wait_quarter(slot, sem):
        for e in range(EQ):
            pltpu.make_async_copy(
                tables_hbm.at[0, 0, pl.ds(0, STRIPE)],
                slot.at[pl.ds(0, STRIPE)], sem).wait()

    def fetch_idx(i):
        pltpu.async_copy(
            x_cat_hbm.at[pl.ds(i * BATCH + col0, BW)], idx_v, idx_sem)

    def wait_idx():
        pltpu.make_async_copy(
            x_cat_hbm.at[pl.ds(0, BW)], idx_v, idx_sem).wait()

    def gather_quarter(slot, q):
        copies = []
        for c in range(EQ * BW // GSUB):
            e = c // (BW // GSUB)
            cc = c % (BW // GSUB)
            copies.append(pltpu.async_copy(
                slot.at[fidx_v.at[pl.ds(e * BW + cc * GSUB, GSUB)]],
                outb_v.at[q * EQ + e, pl.ds(cc * GSUB, GSUB)],
                g_sem))
        for cp in copies:
            cp.wait()

    def drain_write():
        pltpu.make_async_copy(
            tables_hbm.at[0, :, pl.ds(0, BW)], outb_v, wr_sem).wait()

    stage_quarter(f0, 0, slab_a, sem_a)
    stage_quarter(f0, 1, slab_b, sem_b)
    fetch_idx(f0)
    pltpu.async_copy(
        outb_v, out_hbm.at[pl.ds(D_CON + f0 * EMBED_DIM, EMBED_DIM),
                           pl.ds(col0, BW)], wr_sem)
    wait_quarter(slab_a, sem_a)
    plsc.subcore_barrier()

    def do_field(j, carry):
        i = f0 + j
        nxt = jnp.minimum(i + 1, f0 + F_PER_SC - 1)
        wait_idx()

        def build(w, c2):
            v16 = idx_v[pl.ds(w * L, L)]
            for e in range(EQ):
                fidx_v[pl.ds(e * BW + w * L, L)] = v16 + e * VOCAB
            return c2

        lax.fori_loop(0, BW // L, build, 0)
        fetch_idx(nxt)

        drain_write()

        for q in range(N_Q):
            slot = slab_a if q % 2 == 0 else slab_b
            sem = sem_a if q % 2 == 0 else sem_b
            nslot = slab_b if q % 2 == 0 else slab_a
            nsem = sem_b if q % 2 == 0 else sem_a
            gather_quarter(slot, q)
            wait_quarter(nslot, nsem)
            plsc.subcore_barrier()
            if q < N_Q - 2:
                stage_quarter(i, q + 2, slot, sem)
            else:
                stage_quarter(nxt, q + 2 - N_Q, slot, sem)

        pltpu.async_copy(
            outb_v, out_hbm.at[pl.ds(D_CON + i * EMBED_DIM, EMBED_DIM),
                               pl.ds(col0, BW)], wr_sem)
        return carry

    lax.fori_loop(0, F_PER_SC, do_field, 0)

    drain_write()
    wait_idx()
    wait_quarter(slab_b, sem_b)


@jax.jit
def _run(x_con_t, x_cat_flat, tables_t):
    kern = pl.kernel(
        _body,
        out_type=jax.ShapeDtypeStruct((D_OUT, BATCH), jnp.float32),
        mesh=plsc.VectorSubcoreMesh(core_axis_name="c", subcore_axis_name="s"),
        scratch_types=[
            pltpu.VMEM((BW,), jnp.int32),
            pltpu.VMEM((EQ * BW,), jnp.int32),
            pltpu.VMEM((EMBED_DIM, BW), jnp.float32),
            pltpu.VMEM_SHARED((EQ * VOCAB,), jnp.float32),
            pltpu.VMEM_SHARED((EQ * VOCAB,), jnp.float32),
            pltpu.SemaphoreType.DMA,
            pltpu.SemaphoreType.DMA,
            pltpu.SemaphoreType.DMA,
            pltpu.SemaphoreType.DMA,
            pltpu.SemaphoreType.DMA,
        ],
        compiler_params=pltpu.CompilerParams(use_tc_tiling_on_sc=False),
    )
    return kern(x_con_t, x_cat_flat, tables_t)


def kernel(x_con, x_cat, tables):
    x_con_t = x_con.T
    x_cat_flat = x_cat.reshape(N_FIELDS * BATCH)
    tables_t = tables.transpose(0, 2, 1)
    return _run(x_con_t, x_cat_flat, tables_t).T


# speedup vs baseline: 1.0289x; 1.0289x over previous
import functools

import jax
import jax.numpy as jnp
from jax import lax
from jax.experimental import pallas as pl
from jax.experimental.pallas import tpu as pltpu
from jax.experimental.pallas import tpu_sc as plsc

N_FIELDS = 26
VOCAB = 100000
EMBED_DIM = 16
BATCH = 16384
D_CON = 13
D_OUT = D_CON + N_FIELDS * EMBED_DIM

NC, NS, L = 2, 16, 16
F_PER_SC = N_FIELDS // NC
BW = BATCH // NS
STRIPE = 6256
V0_CAP = VOCAB - STRIPE
GSUB = 128
EQ = 4
N_Q = EMBED_DIM // EQ


def _body(x_con_hbm, x_cat_hbm, tables_hbm, out_hbm,
          idx_v, fidx_v, outb_v, slab_a, slab_b,
          sem_a, sem_b, idx_sem, g_sem, wr_sem):
    sc = lax.axis_index("c")
    t = lax.axis_index("s")
    col0 = t * BW
    stripe_v0 = pl.multiple_of(jnp.minimum(t * STRIPE, V0_CAP), 8)
    f0 = sc * F_PER_SC

    pltpu.sync_copy(x_con_hbm.at[:, pl.ds(col0, BW)],
                    outb_v.at[pl.ds(0, D_CON)])
    pltpu.sync_copy(outb_v.at[pl.ds(0, D_CON)],
                    out_hbm.at[pl.ds(0, D_CON), pl.ds(col0, BW)])

    def stage_quarter(i, q, slot, sem):
        for e in range(EQ):
            pltpu.async_copy(
                tables_hbm.at[i, q * EQ + e, pl.ds(stripe_v0, STRIPE)],
                slot.at[pl.ds(e * VOCAB + stripe_v0, STRIPE)],
                sem)

    def wait_quarter(slot, sem):
        for e in range(EQ):
            pltpu.make_async_copy(
                tables_hbm.at[0, 0, pl.ds(0, STRIPE)],
                slot.at[pl.ds(0, STRIPE)], sem).wait()

    def fetch_idx(i):
        pltpu.async_copy(
            x_cat_hbm.at[pl.ds(i * BATCH + col0, BW)], idx_v, idx_sem)

    def wait_idx():
        pltpu.make_async_copy(
            x_cat_hbm.at[pl.ds(0, BW)], idx_v, idx_sem).wait()

    def gather_quarter(slot, q):
        copies = []
        for c in range(EQ * BW // GSUB):
            e = c // (BW // GSUB)
            cc = c % (BW // GSUB)
            copies.append(pltpu.async_copy(
                slot.at[fidx_v.at[pl.ds(e * BW + cc * GSUB, GSUB)]],
                outb_v.at[q * EQ + e, pl.ds(cc * GSUB, GSUB)],
                g_sem))
        for cp in copies:
            cp.wait()

    def drain_write():
        pltpu.make_async_copy(
            tables_hbm.at[0, :, pl.ds(0, BW)], outb_v, wr_sem).wait()

    stage_quarter(f0, 0, slab_a, sem_a)
    stage_quarter(f0, 1, slab_b, sem_b)
    fetch_idx(f0)
    pltpu.async_copy(
        outb_v, out_hbm.at[pl.ds(D_CON + f0 * EMBED_DIM, EMBED_DIM),
                           pl.ds(col0, BW)], wr_sem)

    def do_field(j, carry):
        i = f0 + j
        nxt = jnp.minimum(i + 1, f0 + F_PER_SC - 1)
        wait_idx()

        def build(w, c2):
            v16 = idx_v[pl.ds(w * L, L)]
            for e in range(EQ):
                fidx_v[pl.ds(e * BW + w * L, L)] = v16 + e * VOCAB
            return c2

        lax.fori_loop(0, BW // L, build, 0)
        fetch_idx(nxt)

        drain_write()

        for q in range(N_Q):
            slot = slab_a if q % 2 == 0 else slab_b
            sem = sem_a if q % 2 == 0 else sem_b
            wait_quarter(slot, sem)
            plsc.subcore_barrier()
            gather_quarter(slot, q)
            plsc.subcore_barrier()
            if q < N_Q - 2:
                stage_quarter(i, q + 2, slot, sem)
            else:
                stage_quarter(nxt, q + 2 - N_Q, slot, sem)

        pltpu.async_copy(
            outb_v, out_hbm.at[pl.ds(D_CON + i * EMBED_DIM, EMBED_DIM),
                               pl.ds(col0, BW)], wr_sem)
        return carry

    lax.fori_loop(0, F_PER_SC, do_field, 0)

    drain_write()
    wait_idx()
    wait_quarter(slab_a, sem_a)
    wait_quarter(slab_b, sem_b)


@jax.jit
def _run(x_con_t, x_cat_flat, tables_t):
    kern = pl.kernel(
        _body,
        out_type=jax.ShapeDtypeStruct((D_OUT, BATCH), jnp.float32),
        mesh=plsc.VectorSubcoreMesh(core_axis_name="c", subcore_axis_name="s"),
        scratch_types=[
            pltpu.VMEM((BW,), jnp.int32),
            pltpu.VMEM((EQ * BW,), jnp.int32),
            pltpu.VMEM((EMBED_DIM, BW), jnp.float32),
            pltpu.VMEM_SHARED((EQ * VOCAB,), jnp.float32),
            pltpu.VMEM_SHARED((EQ * VOCAB,), jnp.float32),
            pltpu.SemaphoreType.DMA,
            pltpu.SemaphoreType.DMA,
            pltpu.SemaphoreType.DMA,
            pltpu.SemaphoreType.DMA,
            pltpu.SemaphoreType.DMA,
        ],
        compiler_params=pltpu.CompilerParams(use_tc_tiling_on_sc=False),
    )
    return kern(x_con_t, x_cat_flat, tables_t)


def kernel(x_con, x_cat, tables):
    x_con_t = x_con.T
    x_cat_flat = x_cat.reshape(N_FIELDS * BATCH)
    tables_t = tables.transpose(0, 2, 1)
    return _run(x_con_t, x_cat_flat, tables_t).T


# speedup vs baseline: 1.0300x; 1.0011x over previous
import jax
import jax.numpy as jnp
from jax import lax
from jax.experimental import pallas as pl
from jax.experimental.pallas import tpu as pltpu
from jax.experimental.pallas import tpu_sc as plsc

N_FIELDS = 26
VOCAB = 100000
EMBED_DIM = 16
BATCH = 16384
D_CON = 13
D_OUT = D_CON + N_FIELDS * EMBED_DIM

NC, NS, L = 2, 16, 16
F_PER_SC = N_FIELDS // NC
BW = BATCH // NS
STRIPE = 6256
V0_CAP = VOCAB - STRIPE
GSUB = 128
EQ = 4
N_Q = EMBED_DIM // EQ


def _body(x_con_hbm, x_cat_hbm, tables_hbm, out_hbm,
          idx_v, fidx_v, outb_v, slab_a, slab_b,
          sem_a, sem_b, idx_sem, g_sem, wr_sem):
    sc = lax.axis_index("c")
    t = lax.axis_index("s")
    col0 = t * BW
    stripe_v0 = pl.multiple_of(jnp.minimum(t * STRIPE, V0_CAP), 8)
    f0 = sc * F_PER_SC

    pltpu.sync_copy(x_con_hbm.at[:, pl.ds(col0, BW)],
                    outb_v.at[pl.ds(0, D_CON)])
    pltpu.sync_copy(outb_v.at[pl.ds(0, D_CON)],
                    out_hbm.at[pl.ds(0, D_CON), pl.ds(col0, BW)])

    def stage_quarter(i, q, slot, sem):
        for e in range(EQ):
            pltpu.async_copy(
                tables_hbm.at[i, q * EQ + e, pl.ds(stripe_v0, STRIPE)],
                slot.at[pl.ds(e * VOCAB + stripe_v0, STRIPE)],
                sem)

    def wait_quarter(slot, sem):
        for e in range(EQ):
            pltpu.make_async_copy(
                tables_hbm.at[0, 0, pl.ds(0, STRIPE)],
                slot.at[pl.ds(0, STRIPE)], sem).wait()

    def fetch_idx(i):
        pltpu.async_copy(
            x_cat_hbm.at[pl.ds(i * BATCH + col0, BW)], idx_v, idx_sem)

    def wait_idx():
        pltpu.make_async_copy(
            x_cat_hbm.at[pl.ds(0, BW)], idx_v, idx_sem).wait()

    def gather_quarter(slot, q):
        copies = []
        for c in range(EQ * BW // GSUB):
            e = c // (BW // GSUB)
            cc = c % (BW // GSUB)
            copies.append(pltpu.async_copy(
                slot.at[fidx_v.at[pl.ds(e * BW + cc * GSUB, GSUB)]],
                outb_v.at[q * EQ + e, pl.ds(cc * GSUB, GSUB)],
                g_sem))
        for cp in copies:
            cp.wait()

    def drain_write():
        pltpu.make_async_copy(
            tables_hbm.at[0, :, pl.ds(0, BW)], outb_v, wr_sem).wait()

    stage_quarter(f0, 0, slab_a, sem_a)
    stage_quarter(f0, 1, slab_b, sem_b)
    fetch_idx(f0)
    pltpu.async_copy(
        outb_v, out_hbm.at[pl.ds(D_CON + f0 * EMBED_DIM, EMBED_DIM),
                           pl.ds(col0, BW)], wr_sem)

    def do_field(j, carry):
        i = f0 + j
        nxt = jnp.minimum(i + 1, f0 + F_PER_SC - 1)
        wait_idx()

        def build(w, c2):
            v16 = idx_v[pl.ds(w * L, L)]
            for e in range(EQ):
                fidx_v[pl.ds(e * BW + w * L, L)] = v16 + e * VOCAB
            return c2

        lax.fori_loop(0, BW // L, build, 0)
        fetch_idx(nxt)

        drain_write()

        for q in range(N_Q):
            slot = slab_a if q % 2 == 0 else slab_b
            sem = sem_a if q % 2 == 0 else sem_b
            wait_quarter(slot, sem)
            plsc.subcore_barrier()
            gather_quarter(slot, q)
            plsc.subcore_barrier()
            if q < N_Q - 2:
                stage_quarter(i, q + 2, slot, sem)
            else:
                stage_quarter(nxt, q + 2 - N_Q, slot, sem)

        pltpu.async_copy(
            outb_v, out_hbm.at[pl.ds(D_CON + i * EMBED_DIM, EMBED_DIM),
                               pl.ds(col0, BW)], wr_sem)
        return carry

    lax.fori_loop(0, F_PER_SC, do_field, 0)

    drain_write()
    wait_idx()
    wait_quarter(slab_a, sem_a)
    wait_quarter(slab_b, sem_b)


@jax.jit
def _run(x_con_t, x_cat_flat, tables_t):
    kern = pl.kernel(
        _body,
        out_type=jax.ShapeDtypeStruct((D_OUT, BATCH), jnp.float32),
        mesh=plsc.VectorSubcoreMesh(core_axis_name="c", subcore_axis_name="s"),
        scratch_types=[
            pltpu.VMEM((BW,), jnp.int32),
            pltpu.VMEM((EQ * BW,), jnp.int32),
            pltpu.VMEM((EMBED_DIM, BW), jnp.float32),
            pltpu.VMEM_SHARED((EQ * VOCAB,), jnp.float32),
            pltpu.VMEM_SHARED((EQ * VOCAB,), jnp.float32),
            pltpu.SemaphoreType.DMA,
            pltpu.SemaphoreType.DMA,
            pltpu.SemaphoreType.DMA,
            pltpu.SemaphoreType.DMA,
            pltpu.SemaphoreType.DMA,
        ],
        compiler_params=pltpu.CompilerParams(use_tc_tiling_on_sc=False),
    )
    return kern(x_con_t, x_cat_flat, tables_t)


def kernel(x_con, x_cat, tables):
    x_con_t = x_con.T
    x_cat_flat = x_cat.reshape(N_FIELDS * BATCH)
    tables_t = tables.transpose(0, 2, 1)
    return _run(x_con_t, x_cat_flat, tables_t).T
